# baseline (speedup 1.0000x reference)
import functools

import jax
import jax.numpy as jnp
from jax import lax
from jax.experimental import pallas as pl
from jax.experimental.pallas import tpu as pltpu
from jax.experimental.pallas import tpu_sc as plsc

_L = 16
_D = 64


def _rsqrt_nr(x):
    xi = lax.bitcast_convert_type(x, jnp.int32)
    yi = jnp.int32(0x5F3759DF) - lax.shift_right_logical(xi, 1)
    y = lax.bitcast_convert_type(yi, jnp.float32)
    half = x * 0.5
    for _ in range(3):
        y = y * (1.5 - half * y * y)
    return y


def _allsum(x):
    idx = lax.iota(jnp.int32, _L)
    for s in (8, 4, 2, 1):
        perm = jnp.bitwise_xor(idx, s)
        x = x + jnp.take_along_axis(x, perm, axis=0, mode="promise_in_bounds")
    return x


def _sc_body(d3_hbm, cor_hbm, marg_hbm, table_hbm, out_hbm, div, civ, mv,
             blk0, blk1, blk2, blk3, res_v, sem):
    cid = lax.axis_index("c")
    sid = lax.axis_index("s")

    @pl.when(jnp.logical_and(cid == 0, sid == 0))
    def _():
        stage = [
            pltpu.async_copy(d3_hbm, div.at[pl.ds(0, 3)], sem),
            pltpu.async_copy(cor_hbm, civ.at[pl.ds(0, 1)], sem),
            pltpu.async_copy(marg_hbm, mv.at[pl.ds(0, 1)], sem),
        ]
        for s in stage:
            s.wait()
        vd = div[...]
        vc = civ[...]
        marg = mv[...][0]
        iv = (vd[0], vd[1], vd[2], vc[0])

        blks = (blk0, blk1, blk2, blk3)
        offs = []
        copies = []
        for k in range(4):
            base = pl.multiple_of(jnp.bitwise_and(iv[k], jnp.int32(-128)), 128)
            offs.append(jnp.broadcast_to(iv[k] - base, (_L,)))
            copies.append(
                pltpu.async_copy(
                    table_hbm.at[:, pl.ds(base, 128)], blks[k], sem))
        for c in copies:
            c.wait()

        riota = lax.iota(jnp.int32, _L)

        def col(k, c):
            return plsc.load_gather(blks[k], [riota + c * _L, offs[k]])

        zero = jnp.zeros((_L,), jnp.float32)
        sh, st, sc = zero, zero, zero
        hs, rs, ts, cs = [], [], [], []
        for c in range(_D // _L):
            hc = col(0, c)
            rc = col(1, c)
            tc = col(2, c)
            cc = col(3, c)
            hs.append(hc)
            rs.append(rc)
            ts.append(tc)
            cs.append(cc)
            sh = sh + hc * hc
            st = st + tc * tc
            sc = sc + cc * cc

        big = jnp.float32(1e12)
        inv_h = jnp.minimum(_rsqrt_nr(_allsum(sh)), big)
        inv_t = jnp.minimum(_rsqrt_nr(_allsum(st)), big)
        inv_c = jnp.minimum(_rsqrt_nr(_allsum(sc)), big)

        spos, sneg = zero, zero
        for c in range(_D // _L):
            base = rs[c] - ts[c] * inv_t
            dp = hs[c] * inv_h + base
            dn = cs[c] * inv_c + base
            spos = spos + dp * dp
            sneg = sneg + dn * dn

        pos2 = _allsum(spos)
        neg2 = _allsum(sneg)
        pos = jnp.where(pos2 > 0, pos2 * _rsqrt_nr(pos2), 0.0)
        neg = jnp.where(neg2 > 0, neg2 * _rsqrt_nr(neg2), 0.0)

        res_v[...] = pos - neg + marg
        pltpu.sync_copy(res_v.at[pl.ds(8, 1)], out_hbm)


_sc_kernel = functools.partial(
    pl.kernel,
    out_type=jax.ShapeDtypeStruct((1,), jnp.float32),
    mesh=plsc.VectorSubcoreMesh(
        core_axis_name="c", subcore_axis_name="s", num_cores=1),
    scratch_types=[
        pltpu.VMEM((_L,), jnp.int32),
        pltpu.VMEM((_L,), jnp.int32),
        pltpu.VMEM((_L,), jnp.float32),
        pltpu.VMEM((_D, 128), jnp.float32),
        pltpu.VMEM((_D, 128), jnp.float32),
        pltpu.VMEM((_D, 128), jnp.float32),
        pltpu.VMEM((_D, 128), jnp.float32),
        pltpu.VMEM((_L,), jnp.float32),
        pltpu.SemaphoreType.DMA,
    ],
    compiler_params=pltpu.CompilerParams(
        needs_layout_passes=False, skip_device_barrier=True),
)(_sc_body)


def kernel(data, ent_embeds, corrupt_idx, margin):
    return _sc_kernel(data[-1], corrupt_idx, margin.astype(jnp.float32),
                      ent_embeds.T)


# speedup vs baseline: 1.0079x; 1.0079x over previous
import functools

import jax
import jax.numpy as jnp
from jax import lax
from jax.experimental import pallas as pl
from jax.experimental.pallas import tpu as pltpu
from jax.experimental.pallas import tpu_sc as plsc

_L = 16
_D = 64


def _rsqrt_nr(x):
    xi = lax.bitcast_convert_type(x, jnp.int32)
    yi = jnp.int32(0x5F3759DF) - lax.shift_right_logical(xi, 1)
    y = lax.bitcast_convert_type(yi, jnp.float32)
    half = x * 0.5
    for _ in range(3):
        y = y * (1.5 - half * y * y)
    return y


def _allsum(x):
    idx = lax.iota(jnp.int32, _L)
    for s in (8, 4, 2, 1):
        perm = jnp.bitwise_xor(idx, s)
        x = x + jnp.take_along_axis(x, perm, axis=0, mode="promise_in_bounds")
    return x


def _sc_body(d3_hbm, cor_hbm, marg_hbm, table_hbm, out_hbm, div, civ, mv,
             blk0, blk1, blk2, blk3, res_v, sem):
    cid = lax.axis_index("c")
    sid = lax.axis_index("s")

    @pl.when(jnp.logical_and(cid == 0, sid == 0))
    def _():
        stage = [
            pltpu.async_copy(d3_hbm, div.at[pl.ds(0, 3)], sem),
            pltpu.async_copy(cor_hbm, civ.at[pl.ds(0, 1)], sem),
            pltpu.async_copy(marg_hbm, mv.at[pl.ds(0, 1)], sem),
        ]
        for s in stage:
            s.wait()
        vd = div[...]
        vc = civ[...]
        marg = mv[...][0]
        iv = (vd[0], vd[1], vd[2], vc[0])

        blks = (blk0, blk1, blk2, blk3)
        offs = []
        copies = []
        for k in range(4):
            base = pl.multiple_of(jnp.bitwise_and(iv[k], jnp.int32(-128)), 128)
            offs.append(jnp.broadcast_to(iv[k] - base, (_L,)))
            copies.append(
                pltpu.async_copy(
                    table_hbm.at[:, pl.ds(base, 128)], blks[k], sem))
        for c in copies:
            c.wait()

        riota = lax.iota(jnp.int32, _L)

        def col(k, c):
            return plsc.load_gather(blks[k], [riota + c * _L, offs[k]])

        zero = jnp.zeros((_L,), jnp.float32)
        sh, st, sc = zero, zero, zero
        hs, rs, ts, cs = [], [], [], []
        for c in range(_D // _L):
            hc = col(0, c)
            rc = col(1, c)
            tc = col(2, c)
            cc = col(3, c)
            hs.append(hc)
            rs.append(rc)
            ts.append(tc)
            cs.append(cc)
            sh = sh + hc * hc
            st = st + tc * tc
            sc = sc + cc * cc

        big = jnp.float32(1e12)
        inv_h = jnp.minimum(_rsqrt_nr(_allsum(sh)), big)
        inv_t = jnp.minimum(_rsqrt_nr(_allsum(st)), big)
        inv_c = jnp.minimum(_rsqrt_nr(_allsum(sc)), big)

        spos, sneg = zero, zero
        for c in range(_D // _L):
            base = rs[c] - ts[c] * inv_t
            dp = hs[c] * inv_h + base
            dn = cs[c] * inv_c + base
            spos = spos + dp * dp
            sneg = sneg + dn * dn

        pos2 = _allsum(spos)
        neg2 = _allsum(sneg)
        pos = jnp.where(pos2 > 0, pos2 * _rsqrt_nr(pos2), 0.0)
        neg = jnp.where(neg2 > 0, neg2 * _rsqrt_nr(neg2), 0.0)

        res_v[...] = pos - neg + marg
        pltpu.sync_copy(res_v.at[pl.ds(8, 1)], out_hbm)


_sc_kernel = functools.partial(
    pl.kernel,
    out_type=jax.ShapeDtypeStruct((1,), jnp.float32),
    mesh=plsc.VectorSubcoreMesh(
        core_axis_name="c", subcore_axis_name="s", num_cores=1,
        num_subcores=1),
    scratch_types=[
        pltpu.VMEM((_L,), jnp.int32),
        pltpu.VMEM((_L,), jnp.int32),
        pltpu.VMEM((_L,), jnp.float32),
        pltpu.VMEM((_D, 128), jnp.float32),
        pltpu.VMEM((_D, 128), jnp.float32),
        pltpu.VMEM((_D, 128), jnp.float32),
        pltpu.VMEM((_D, 128), jnp.float32),
        pltpu.VMEM((_L,), jnp.float32),
        pltpu.SemaphoreType.DMA,
    ],
    compiler_params=pltpu.CompilerParams(needs_layout_passes=False),
)(_sc_body)


def kernel(data, ent_embeds, corrupt_idx, margin):
    return _sc_kernel(data[-1], corrupt_idx, margin.astype(jnp.float32),
                      ent_embeds.T)
